# baseline (speedup 1.0000x reference)
import functools

import jax
import jax.numpy as jnp
from jax import lax
from jax.experimental import pallas as pl
from jax.experimental.pallas import tpu as pltpu
from jax.experimental.pallas import tpu_sc as plsc

N = 10000
E = 320000
D = 128

NC = 2
NS = 16
EPT = E // NS
C = 128
EPT_PAD = 20480
NCHUNK = EPT_PAD // C
NBUF = 2
IB = 8
NBLK = NCHUNK // IB
NBLK_PAD = NBLK + 2
NPAD = 10016
ZROWS = NPAD // NS
H = D // 2
DW = 8


@functools.cache
def _make_sc_agg(with_deg):
    mesh = plsc.VectorSubcoreMesh(
        core_axis_name="c", subcore_axis_name="s",
        num_cores=NC, num_subcores=NS)

    out_type = [jax.ShapeDtypeStruct((NC, NPAD, H), jnp.float32)]
    scratch = [
        [pltpu.VMEM((2, IB, C), jnp.int32) for _ in range(2)],
        [pltpu.VMEM((C, H), jnp.float32) for _ in range(NBUF)],
        pltpu.VMEM_SHARED((NPAD, H), jnp.float32),
        pltpu.VMEM_SHARED((NPAD, H), jnp.float32),
        [pltpu.SemaphoreType.DMA for _ in range(NBUF)],
        [pltpu.SemaphoreType.DMA for _ in range(2)],
    ]
    if with_deg:
        out_type.append(jax.ShapeDtypeStruct((NPAD, DW), jnp.float32))
        scratch.append(pltpu.VMEM((C, DW), jnp.float32))
        scratch.append(pltpu.VMEM_SHARED((NPAD, DW), jnp.float32))

    @functools.partial(
        pl.kernel,
        out_type=out_type,
        mesh=mesh,
        scratch_types=scratch,
        compiler_params=pltpu.CompilerParams(use_tc_tiling_on_sc=False),
    )
    def sc_agg(table, edp, zeros, *rest):
        if with_deg:
            (ones_c, zeros_deg, out, out_deg,
             ed_blk, rows, tab_sh, agg_sh, gsem, isem, ones_v, deg_sh) = rest
        else:
            out, ed_blk, rows, tab_sh, agg_sh, gsem, isem = rest
        c = lax.axis_index("c")
        s = lax.axis_index("s")

        def load_idx(blk, par):
            pltpu.async_copy(edp.at[s, blk], ed_blk[par], isem[par])

        def wait_idx(par):
            pltpu.make_async_copy(edp.at[s, 0], ed_blk[par],
                                  isem[par]).wait()

        rs = pl.ds(s * ZROWS, ZROWS)
        pltpu.sync_copy(table.at[c, rs], tab_sh.at[rs])
        pltpu.sync_copy(zeros, agg_sh.at[rs])
        if with_deg:
            @pl.when(c == 0)
            def _():
                pltpu.sync_copy(ones_c, ones_v)
                pltpu.sync_copy(zeros_deg, deg_sh.at[rs])
        load_idx(0, 0)
        load_idx(1, 1)
        wait_idx(0)
        plsc.subcore_barrier()
        for k in range(NBUF):
            pltpu.async_copy(tab_sh.at[ed_blk[0].at[0, k]], rows[k], gsem[k])

        def body(bi, carry):
            b0 = 2 * bi
            for par in range(2):
                blk = b0 + par
                for k in range(IB):
                    p = k % NBUF
                    if k == IB - NBUF:
                        wait_idx(1 - par)
                    pltpu.make_async_copy(tab_sh.at[ed_blk[par].at[0, k]],
                                          rows[p], gsem[p]).wait()
                    pltpu.sync_copy(rows[p], agg_sh.at[ed_blk[par].at[1, k]],
                                    add=True)
                    if with_deg:
                        @pl.when(c == 0)
                        def _():
                            pltpu.sync_copy(
                                ones_v, deg_sh.at[ed_blk[par].at[1, k]],
                                add=True)
                    if k < IB - NBUF:
                        nxt = tab_sh.at[ed_blk[par].at[0, k + NBUF]]
                    else:
                        nxt = tab_sh.at[ed_blk[1 - par].at[0, k - (IB - NBUF)]]
                    pltpu.async_copy(nxt, rows[p], gsem[p])
                load_idx(blk + 2, par)
            return carry

        lax.fori_loop(0, NBLK // 2, body, 0)

        for k in range(NBUF):
            pltpu.make_async_copy(tab_sh.at[ed_blk[0].at[0, 0]], rows[k],
                                  gsem[k]).wait()
        wait_idx(1)

        plsc.subcore_barrier()
        pltpu.sync_copy(agg_sh.at[rs], out.at[c, rs])
        if with_deg:
            @pl.when(c == 0)
            def _():
                pltpu.sync_copy(deg_sh.at[rs], out_deg.at[rs])

    return sc_agg


BM = 2504


def _tc0_body(p_ref, degp_ref, x_ref, wl_ref, wr_ref, b_ref,
              h1_ref, h1s_ref, invd_ref):
    agg = jnp.concatenate([p_ref[0], p_ref[1]], axis=1)
    deg = degp_ref[:, 0:1]
    invd = 1.0 / jnp.maximum(deg, 1.0)
    mean = agg * invd
    h = jnp.dot(mean, wl_ref[...], preferred_element_type=jnp.float32)
    h = h + jnp.dot(x_ref[...], wr_ref[...], preferred_element_type=jnp.float32)
    h = h + b_ref[...]
    h = jnp.maximum(h, 0.0)
    h1_ref[...] = h
    h1s_ref[0] = h[:, :H]
    h1s_ref[1] = h[:, H:]
    invd_ref[...] = invd


def _tc1_body(p_ref, h1_ref, invd_ref, wl_ref, wr_ref, b_ref, out_ref):
    agg = jnp.concatenate([p_ref[0], p_ref[1]], axis=1)
    mean = agg * invd_ref[...]
    o = jnp.dot(mean, wl_ref[...], preferred_element_type=jnp.float32)
    o = o + jnp.dot(h1_ref[...], wr_ref[...], preferred_element_type=jnp.float32)
    out_ref[...] = o + b_ref[...]


_GRID = NPAD // BM
_W_SPEC = pl.BlockSpec((D, D), lambda i: (0, 0))
_B_SPEC = pl.BlockSpec((1, D), lambda i: (0, 0))

_tc0 = pl.pallas_call(
    _tc0_body,
    grid=(_GRID,),
    in_specs=[
        pl.BlockSpec((NC, BM, H), lambda i: (0, i, 0)),
        pl.BlockSpec((BM, DW), lambda i: (i, 0)),
        pl.BlockSpec((BM, D), lambda i: (i, 0)),
        _W_SPEC, _W_SPEC, _B_SPEC,
    ],
    out_specs=[
        pl.BlockSpec((BM, D), lambda i: (i, 0)),
        pl.BlockSpec((NC, BM, H), lambda i: (0, i, 0)),
        pl.BlockSpec((BM, 1), lambda i: (i, 0)),
    ],
    out_shape=[
        jax.ShapeDtypeStruct((NPAD, D), jnp.float32),
        jax.ShapeDtypeStruct((NC, NPAD, H), jnp.float32),
        jax.ShapeDtypeStruct((NPAD, 1), jnp.float32),
    ],
)

_tc1 = pl.pallas_call(
    _tc1_body,
    grid=(_GRID,),
    in_specs=[
        pl.BlockSpec((NC, BM, H), lambda i: (0, i, 0)),
        pl.BlockSpec((BM, D), lambda i: (i, 0)),
        pl.BlockSpec((BM, 1), lambda i: (i, 0)),
        _W_SPEC, _W_SPEC, _B_SPEC,
    ],
    out_specs=pl.BlockSpec((BM, D), lambda i: (i, 0)),
    out_shape=jax.ShapeDtypeStruct((N, D), jnp.float32),
)


def kernel(x, W_l0, W_r0, b0, W_l1, W_r1, b1, edge_index):
    src = edge_index[0].reshape(NS, EPT)
    dst = edge_index[1].reshape(NS, EPT)
    srcp = jnp.pad(src, ((0, 0), (0, EPT_PAD - EPT)))
    srcp = jnp.pad(srcp.reshape(NS, NBLK, IB, C),
                   ((0, 0), (0, 2), (0, 0), (0, 0)))
    dstp = jnp.pad(dst, ((0, 0), (0, EPT_PAD - EPT)), constant_values=N)
    dstp = jnp.pad(dstp.reshape(NS, NBLK, IB, C),
                   ((0, 0), (0, 2), (0, 0), (0, 0)), constant_values=N)
    edp = jnp.stack([srcp, dstp], axis=2)
    x_pad = jnp.pad(x, ((0, NPAD - N), (0, 0)))
    x_halves = jnp.stack([x_pad[:, :H], x_pad[:, H:]])
    zeros = jnp.zeros((ZROWS, H), jnp.float32)
    ones_c = jnp.ones((C, DW), jnp.float32)
    zeros_deg = jnp.zeros((ZROWS, DW), jnp.float32)

    p0, deg = _make_sc_agg(True)(x_halves, edp, zeros, ones_c, zeros_deg)
    h1, h1_halves, invd = _tc0(p0, deg, x_pad, W_l0, W_r0, b0.reshape(1, D))
    p1, = _make_sc_agg(False)(h1_halves, edp, zeros)
    out = _tc1(p1, h1, invd, W_l1, W_r1, b1.reshape(1, D))
    return out


# speedup vs baseline: 1.0189x; 1.0189x over previous
import functools

import jax
import jax.numpy as jnp
from jax import lax
from jax.experimental import pallas as pl
from jax.experimental.pallas import tpu as pltpu
from jax.experimental.pallas import tpu_sc as plsc

N = 10000
E = 320000
D = 128

NC = 2
NS = 16
EPT = E // NS
C = 128
EPT_PAD = 20480
NCHUNK = EPT_PAD // C
NBUF = 2
IB = 8
NBLK = NCHUNK // IB
NBLK_PAD = NBLK + 2
NPAD = 10016
ZROWS = NPAD // NS
H = D // 2
DW = 8


@functools.cache
def _make_sc_agg(with_deg):
    mesh = plsc.VectorSubcoreMesh(
        core_axis_name="c", subcore_axis_name="s",
        num_cores=NC, num_subcores=NS)

    out_type = [jax.ShapeDtypeStruct((NC, NPAD, H), jnp.float32)]
    scratch = [
        [pltpu.VMEM((2, IB, C), jnp.int32) for _ in range(2)],
        [pltpu.VMEM((C, H), jnp.float32) for _ in range(NBUF)],
        pltpu.VMEM_SHARED((NPAD, H), jnp.float32),
        pltpu.VMEM_SHARED((NPAD, H), jnp.float32),
        [pltpu.SemaphoreType.DMA for _ in range(NBUF)],
        [pltpu.SemaphoreType.DMA for _ in range(2)],
    ]
    if with_deg:
        out_type.append(jax.ShapeDtypeStruct((NPAD, DW), jnp.float32))
        scratch.append(pltpu.VMEM((C, DW), jnp.float32))
        scratch.append(pltpu.VMEM_SHARED((NPAD, DW), jnp.float32))
        scratch.append(pltpu.SemaphoreType.DMA)

    @functools.partial(
        pl.kernel,
        out_type=out_type,
        mesh=mesh,
        scratch_types=scratch,
        compiler_params=pltpu.CompilerParams(use_tc_tiling_on_sc=False),
    )
    def sc_agg(table, edp, zeros, *rest):
        if with_deg:
            (ones_c, zeros_deg, out, out_deg, ed_blk, rows,
             tab_sh, agg_sh, gsem, isem, ones_v, deg_sh, dsem) = rest
        else:
            out, ed_blk, rows, tab_sh, agg_sh, gsem, isem = rest
        c = lax.axis_index("c")
        s = lax.axis_index("s")

        def load_idx(blk, par):
            pltpu.async_copy(edp.at[s, blk], ed_blk[par], isem[par])

        def wait_idx(par):
            pltpu.make_async_copy(edp.at[s, 0], ed_blk[par],
                                  isem[par]).wait()

        rs = pl.ds(s * ZROWS, ZROWS)
        pltpu.sync_copy(table.at[c, rs], tab_sh.at[rs])
        pltpu.sync_copy(zeros, agg_sh.at[rs])
        if with_deg:
            @pl.when(c == 0)
            def _():
                pltpu.sync_copy(ones_c, ones_v)
                pltpu.sync_copy(zeros_deg, deg_sh.at[rs])
        load_idx(0, 0)
        load_idx(1, 1)
        wait_idx(0)
        plsc.subcore_barrier()
        for k in range(NBUF):
            pltpu.async_copy(tab_sh.at[ed_blk[0].at[0, k]], rows[k], gsem[k])

        def body(bi, carry):
            b0 = 2 * bi
            for par in range(2):
                blk = b0 + par
                for k in range(IB):
                    p = k % NBUF
                    if k == IB - NBUF:
                        wait_idx(1 - par)
                    pltpu.make_async_copy(tab_sh.at[ed_blk[par].at[0, k]],
                                          rows[p], gsem[p]).wait()
                    pltpu.sync_copy(rows[p], agg_sh.at[ed_blk[par].at[1, k]],
                                    add=True)
                    if with_deg:
                        @pl.when(c == 0)
                        def _():
                            pltpu.async_copy(
                                ones_v, deg_sh.at[ed_blk[par].at[1, k]],
                                dsem, add=True)
                    if k < IB - NBUF:
                        nxt = tab_sh.at[ed_blk[par].at[0, k + NBUF]]
                    else:
                        nxt = tab_sh.at[ed_blk[1 - par].at[0, k - (IB - NBUF)]]
                    pltpu.async_copy(nxt, rows[p], gsem[p])
                load_idx(blk + 2, par)
            return carry

        lax.fori_loop(0, NBLK // 2, body, 0)

        for k in range(NBUF):
            pltpu.make_async_copy(tab_sh.at[ed_blk[0].at[0, 0]], rows[k],
                                  gsem[k]).wait()
        wait_idx(1)
        if with_deg:
            @pl.when(c == 0)
            def _():
                def dwait(i, carry):
                    pltpu.make_async_copy(
                        ones_v, deg_sh.at[ed_blk[0].at[1, 0]], dsem).wait()
                    return carry
                lax.fori_loop(0, NCHUNK, dwait, 0)

        plsc.subcore_barrier()
        pltpu.sync_copy(agg_sh.at[rs], out.at[c, rs])
        if with_deg:
            @pl.when(c == 0)
            def _():
                pltpu.sync_copy(deg_sh.at[rs], out_deg.at[rs])

    return sc_agg


BM = 2504


def _tc0_body(p_ref, degp_ref, x_ref, wl_ref, wr_ref, b_ref,
              h1_ref, h1s_ref, invd_ref):
    agg = jnp.concatenate([p_ref[0], p_ref[1]], axis=1)
    deg = degp_ref[:, 0:1]
    invd = 1.0 / jnp.maximum(deg, 1.0)
    mean = agg * invd
    h = jnp.dot(mean, wl_ref[...], preferred_element_type=jnp.float32)
    h = h + jnp.dot(x_ref[...], wr_ref[...], preferred_element_type=jnp.float32)
    h = h + b_ref[...]
    h = jnp.maximum(h, 0.0)
    h1_ref[...] = h
    h1s_ref[0] = h[:, :H]
    h1s_ref[1] = h[:, H:]
    invd_ref[...] = invd


def _tc1_body(p_ref, h1_ref, invd_ref, wl_ref, wr_ref, b_ref, out_ref):
    agg = jnp.concatenate([p_ref[0], p_ref[1]], axis=1)
    mean = agg * invd_ref[...]
    o = jnp.dot(mean, wl_ref[...], preferred_element_type=jnp.float32)
    o = o + jnp.dot(h1_ref[...], wr_ref[...], preferred_element_type=jnp.float32)
    out_ref[...] = o + b_ref[...]


_GRID = NPAD // BM
_W_SPEC = pl.BlockSpec((D, D), lambda i: (0, 0))
_B_SPEC = pl.BlockSpec((1, D), lambda i: (0, 0))

_tc0 = pl.pallas_call(
    _tc0_body,
    grid=(_GRID,),
    in_specs=[
        pl.BlockSpec((NC, BM, H), lambda i: (0, i, 0)),
        pl.BlockSpec((BM, DW), lambda i: (i, 0)),
        pl.BlockSpec((BM, D), lambda i: (i, 0)),
        _W_SPEC, _W_SPEC, _B_SPEC,
    ],
    out_specs=[
        pl.BlockSpec((BM, D), lambda i: (i, 0)),
        pl.BlockSpec((NC, BM, H), lambda i: (0, i, 0)),
        pl.BlockSpec((BM, 1), lambda i: (i, 0)),
    ],
    out_shape=[
        jax.ShapeDtypeStruct((NPAD, D), jnp.float32),
        jax.ShapeDtypeStruct((NC, NPAD, H), jnp.float32),
        jax.ShapeDtypeStruct((NPAD, 1), jnp.float32),
    ],
)

_tc1 = pl.pallas_call(
    _tc1_body,
    grid=(_GRID,),
    in_specs=[
        pl.BlockSpec((NC, BM, H), lambda i: (0, i, 0)),
        pl.BlockSpec((BM, D), lambda i: (i, 0)),
        pl.BlockSpec((BM, 1), lambda i: (i, 0)),
        _W_SPEC, _W_SPEC, _B_SPEC,
    ],
    out_specs=pl.BlockSpec((BM, D), lambda i: (i, 0)),
    out_shape=jax.ShapeDtypeStruct((N, D), jnp.float32),
)


def kernel(x, W_l0, W_r0, b0, W_l1, W_r1, b1, edge_index):
    src = edge_index[0].reshape(NS, EPT)
    dst = edge_index[1].reshape(NS, EPT)
    srcp = jnp.pad(src, ((0, 0), (0, EPT_PAD - EPT)))
    srcp = jnp.pad(srcp.reshape(NS, NBLK, IB, C),
                   ((0, 0), (0, 2), (0, 0), (0, 0)))
    dstp = jnp.pad(dst, ((0, 0), (0, EPT_PAD - EPT)), constant_values=N)
    dstp = jnp.pad(dstp.reshape(NS, NBLK, IB, C),
                   ((0, 0), (0, 2), (0, 0), (0, 0)), constant_values=N)
    edp = jnp.stack([srcp, dstp], axis=2)
    x_pad = jnp.pad(x, ((0, NPAD - N), (0, 0)))
    x_halves = jnp.stack([x_pad[:, :H], x_pad[:, H:]])
    zeros = jnp.zeros((ZROWS, H), jnp.float32)
    ones_c = jnp.ones((C, DW), jnp.float32)
    zeros_deg = jnp.zeros((ZROWS, DW), jnp.float32)

    p0, deg = _make_sc_agg(True)(x_halves, edp, zeros, ones_c, zeros_deg)
    h1, h1_halves, invd = _tc0(p0, deg, x_pad, W_l0, W_r0, b0.reshape(1, D))
    p1, = _make_sc_agg(False)(h1_halves, edp, zeros)
    out = _tc1(p1, h1, invd, W_l1, W_r1, b1.reshape(1, D))
    return out


# speedup vs baseline: 1.1172x; 1.0965x over previous
import functools

import jax
import jax.numpy as jnp
from jax import lax
from jax.experimental import pallas as pl
from jax.experimental.pallas import tpu as pltpu
from jax.experimental.pallas import tpu_sc as plsc

N = 10000
E = 320000
D = 128

NC = 2
NS = 16
EPT = E // NS
C = 128
EPT_PAD = 20480
NCHUNK = EPT_PAD // C
NBUF = 2
IB = 8
NBLK = NCHUNK // IB
NBLK_PAD = NBLK + 2
NPAD = 10016
ZROWS = NPAD // NS
H = D // 2
DW = 8


@functools.cache
def _make_sc_agg(with_deg):
    mesh = plsc.VectorSubcoreMesh(
        core_axis_name="c", subcore_axis_name="s",
        num_cores=NC, num_subcores=NS)

    out_type = [jax.ShapeDtypeStruct((NC, NPAD, H), jnp.float32)]
    scratch = [
        [pltpu.VMEM((2, IB, C), jnp.int32) for _ in range(2)],
        [pltpu.VMEM((C, H), jnp.float32) for _ in range(NBUF)],
        pltpu.VMEM_SHARED((NPAD, H), jnp.float32),
        pltpu.VMEM_SHARED((NPAD, H), jnp.float32),
        [pltpu.SemaphoreType.DMA for _ in range(NBUF)],
        [pltpu.SemaphoreType.DMA for _ in range(2)],
    ]
    if with_deg:
        out_type.append(jax.ShapeDtypeStruct((NC, NPAD, DW), jnp.float32))
        scratch.append(pltpu.VMEM((C, DW), jnp.float32))
        scratch.append(pltpu.VMEM_SHARED((NPAD, DW), jnp.float32))
        scratch.append(pltpu.SemaphoreType.DMA)

    @functools.partial(
        pl.kernel,
        out_type=out_type,
        mesh=mesh,
        scratch_types=scratch,
        compiler_params=pltpu.CompilerParams(use_tc_tiling_on_sc=False),
    )
    def sc_agg(table, edp, zeros, *rest):
        if with_deg:
            (ones_c, zeros_deg, out, out_deg, ed_blk, rows,
             tab_sh, agg_sh, gsem, isem, ones_v, deg_sh, dsem) = rest
        else:
            out, ed_blk, rows, tab_sh, agg_sh, gsem, isem = rest
        c = lax.axis_index("c")
        s = lax.axis_index("s")

        def load_idx(blk, par):
            pltpu.async_copy(edp.at[s, blk], ed_blk[par], isem[par])

        def wait_idx(par):
            pltpu.make_async_copy(edp.at[s, 0], ed_blk[par],
                                  isem[par]).wait()

        rs = pl.ds(s * ZROWS, ZROWS)
        pltpu.sync_copy(table.at[rs, pl.ds(c * H, H)], tab_sh.at[rs])
        pltpu.sync_copy(zeros, agg_sh.at[rs])
        if with_deg:
            pltpu.sync_copy(ones_c, ones_v)
            pltpu.sync_copy(zeros_deg, deg_sh.at[rs])
        load_idx(0, 0)
        load_idx(1, 1)
        wait_idx(0)
        plsc.subcore_barrier()
        for k in range(NBUF):
            pltpu.async_copy(tab_sh.at[ed_blk[0].at[0, k]], rows[k], gsem[k])

        def body(bi, carry):
            b0 = 2 * bi
            for par in range(2):
                blk = b0 + par
                for k in range(IB):
                    p = k % NBUF
                    if k == IB - NBUF:
                        wait_idx(1 - par)
                    pltpu.make_async_copy(tab_sh.at[ed_blk[par].at[0, k]],
                                          rows[p], gsem[p]).wait()
                    pltpu.sync_copy(rows[p], agg_sh.at[ed_blk[par].at[1, k]],
                                    add=True)
                    if with_deg:
                        @pl.when(c == par)
                        def _():
                            pltpu.async_copy(
                                ones_v, deg_sh.at[ed_blk[par].at[1, k]],
                                dsem, add=True)
                    if k < IB - NBUF:
                        nxt = tab_sh.at[ed_blk[par].at[0, k + NBUF]]
                    else:
                        nxt = tab_sh.at[ed_blk[1 - par].at[0, k - (IB - NBUF)]]
                    pltpu.async_copy(nxt, rows[p], gsem[p])
                load_idx(blk + 2, par)
            return carry

        lax.fori_loop(0, NBLK // 2, body, 0)

        for k in range(NBUF):
            pltpu.make_async_copy(tab_sh.at[ed_blk[0].at[0, 0]], rows[k],
                                  gsem[k]).wait()
        wait_idx(1)
        if with_deg:
            def dwait(i, carry):
                pltpu.make_async_copy(
                    ones_v, deg_sh.at[ed_blk[0].at[1, 0]], dsem).wait()
                return carry
            lax.fori_loop(0, NCHUNK // 2, dwait, 0)

        plsc.subcore_barrier()
        pltpu.sync_copy(agg_sh.at[rs], out.at[c, rs])
        if with_deg:
            pltpu.sync_copy(deg_sh.at[rs], out_deg.at[c, rs])

    return sc_agg


BM = 2504


def _tc0_body(p_ref, degp_ref, x_ref, wl_ref, wr_ref, b_ref,
              h1_ref, invd_ref):
    agg = jnp.concatenate([p_ref[0], p_ref[1]], axis=1)
    deg = degp_ref[0, :, 0:1] + degp_ref[1, :, 0:1]
    invd = 1.0 / jnp.maximum(deg, 1.0)
    mean = agg * invd
    h = jnp.dot(mean, wl_ref[...], preferred_element_type=jnp.float32)
    h = h + jnp.dot(x_ref[...], wr_ref[...], preferred_element_type=jnp.float32)
    h = h + b_ref[...]
    h1_ref[...] = jnp.maximum(h, 0.0)
    invd_ref[...] = invd


def _tc1_body(p_ref, h1_ref, invd_ref, wl_ref, wr_ref, b_ref, out_ref):
    agg = jnp.concatenate([p_ref[0], p_ref[1]], axis=1)
    mean = agg * invd_ref[...]
    o = jnp.dot(mean, wl_ref[...], preferred_element_type=jnp.float32)
    o = o + jnp.dot(h1_ref[...], wr_ref[...], preferred_element_type=jnp.float32)
    out_ref[...] = o + b_ref[...]


_GRID = NPAD // BM
_W_SPEC = pl.BlockSpec((D, D), lambda i: (0, 0))
_B_SPEC = pl.BlockSpec((1, D), lambda i: (0, 0))

_tc0 = pl.pallas_call(
    _tc0_body,
    grid=(_GRID,),
    in_specs=[
        pl.BlockSpec((NC, BM, H), lambda i: (0, i, 0)),
        pl.BlockSpec((NC, BM, DW), lambda i: (0, i, 0)),
        pl.BlockSpec((BM, D), lambda i: (i, 0)),
        _W_SPEC, _W_SPEC, _B_SPEC,
    ],
    out_specs=[
        pl.BlockSpec((BM, D), lambda i: (i, 0)),
        pl.BlockSpec((BM, 1), lambda i: (i, 0)),
    ],
    out_shape=[
        jax.ShapeDtypeStruct((NPAD, D), jnp.float32),
        jax.ShapeDtypeStruct((NPAD, 1), jnp.float32),
    ],
)

_tc1 = pl.pallas_call(
    _tc1_body,
    grid=(_GRID,),
    in_specs=[
        pl.BlockSpec((NC, BM, H), lambda i: (0, i, 0)),
        pl.BlockSpec((BM, D), lambda i: (i, 0)),
        pl.BlockSpec((BM, 1), lambda i: (i, 0)),
        _W_SPEC, _W_SPEC, _B_SPEC,
    ],
    out_specs=pl.BlockSpec((BM, D), lambda i: (i, 0)),
    out_shape=jax.ShapeDtypeStruct((N, D), jnp.float32),
)


def kernel(x, W_l0, W_r0, b0, W_l1, W_r1, b1, edge_index):
    src = edge_index[0].reshape(NS, EPT)
    dst = edge_index[1].reshape(NS, EPT)
    srcp = jnp.pad(src, ((0, 0), (0, EPT_PAD - EPT)))
    srcp = jnp.pad(srcp.reshape(NS, NBLK, IB, C),
                   ((0, 0), (0, 2), (0, 0), (0, 0)))
    dstp = jnp.pad(dst, ((0, 0), (0, EPT_PAD - EPT)), constant_values=N)
    dstp = jnp.pad(dstp.reshape(NS, NBLK, IB, C),
                   ((0, 0), (0, 2), (0, 0), (0, 0)), constant_values=N)
    edp = jnp.stack([srcp, dstp], axis=2)
    x_pad = jnp.pad(x, ((0, NPAD - N), (0, 0)))
    zeros = jnp.zeros((ZROWS, H), jnp.float32)
    ones_c = jnp.ones((C, DW), jnp.float32)
    zeros_deg = jnp.zeros((ZROWS, DW), jnp.float32)

    p0, deg = _make_sc_agg(True)(x_pad, edp, zeros, ones_c, zeros_deg)
    h1, invd = _tc0(p0, deg, x_pad, W_l0, W_r0, b0.reshape(1, D))
    p1, = _make_sc_agg(False)(h1, edp, zeros)
    out = _tc1(p1, h1, invd, W_l1, W_r1, b1.reshape(1, D))
    return out
